# baseline (speedup 1.0000x reference)
import jax
import jax.numpy as jnp
from jax import lax
from jax.experimental import pallas as pl
from jax.experimental.pallas import tpu as pltpu
from jax.experimental.pallas import tpu_sc as plsc

N = 10000
E = 320000
D = 128
NC, NS = 2, 16
NW = NC * NS
CB = 128
CH = 81
NBUF = 3
EP = NW * CH * CB
RPT = 632
NP = NS * RPT

_MESH = plsc.VectorSubcoreMesh(
    core_axis_name="c", subcore_axis_name="s", num_cores=NC, num_subcores=NS
)


def _hist_body(ei_ref, hs_ref, hd_ref, idx_v, hist_v):
    c = lax.axis_index("c")
    s = lax.axis_index("s")

    def zero(i, carry):
        hist_v[pl.ds(i * 16, 16)] = jnp.zeros((16,), jnp.float32)
        return carry

    lax.fori_loop(0, NP // 16, zero, 0)
    pltpu.sync_copy(ei_ref.at[c, s], idx_v)
    ones = jnp.ones((16,), jnp.float32)

    def chunk(j, carry):
        for k in range(CB // 16):
            idx16 = idx_v[j, pl.ds(k * 16, 16)]
            plsc.addupdate_scatter(hist_v, [idx16], ones)
        return carry

    lax.fori_loop(0, 2 * CH, chunk, 0)

    @pl.when(c == 0)
    def _():
        pltpu.sync_copy(hist_v, hs_ref.at[s])

    @pl.when(c == 1)
    def _():
        pltpu.sync_copy(hist_v, hd_ref.at[s])


_hist_kernel = pl.kernel(
    _hist_body,
    out_type=(
        jax.ShapeDtypeStruct((NS, NP), jnp.float32),
        jax.ShapeDtypeStruct((NS, NP), jnp.float32),
    ),
    mesh=_MESH,
    scratch_types=[
        pltpu.VMEM((2 * CH, CB), jnp.int32),
        pltpu.VMEM((NP,), jnp.float32),
    ],
    compiler_params=pltpu.CompilerParams(needs_layout_passes=False),
)


def _mm_body(x_ref, w_ref, hs_ref, hd_ref, y_ref, nrm_ref):
    ones_col = jnp.ones((NS, 1), jnp.float32)
    dn = (((0,), (0,)), ((), ()))
    outdeg = lax.dot_general(hs_ref[...], ones_col, dn,
                             preferred_element_type=jnp.float32)
    nsrc = lax.rsqrt(jnp.maximum(outdeg, 1.0))
    h = x_ref[...] * nsrc
    y_ref[...] = jnp.dot(h, w_ref[...], preferred_element_type=jnp.float32)
    indeg = lax.dot_general(hd_ref[...], ones_col, dn,
                            preferred_element_type=jnp.float32)
    ndst = lax.rsqrt(jnp.maximum(indeg, 1.0))
    nrm_ref[...] = jnp.broadcast_to(ndst, (NP, 8))


_mm_kernel = pl.pallas_call(
    _mm_body,
    out_shape=(
        jax.ShapeDtypeStruct((NP, D), jnp.float32),
        jax.ShapeDtypeStruct((NP, 8), jnp.float32),
    ),
)


def _mp_body(ei_ref, y_ref, zeros_ref, p0_ref, p1_ref,
             idx_r, b0, b1, b2, acc_sh, gsem, isem):
    c = lax.axis_index("c")
    s = lax.axis_index("s")
    q = c * NS + s
    rows = pl.ds(s * RPT, RPT)
    pltpu.sync_copy(zeros_ref.at[rows], acc_sh.at[rows])
    plsc.subcore_barrier()

    bufs = (b0, b1, b2)

    def l_start(j, m):
        pltpu.async_copy(ei_ref.at[q, j], idx_r.at[m], isem)

    def l_wait(j, m):
        pltpu.make_async_copy(ei_ref.at[q, j], idx_r.at[m], isem).wait()

    def g_start(j, m):
        pltpu.async_copy(y_ref.at[idx_r.at[m, 0]], bufs[m], gsem)

    def g_wait(j, m):
        pltpu.make_async_copy(y_ref.at[idx_r.at[m, 0]], bufs[m], gsem).wait()

    for m in range(NBUF):
        l_start(m, m)
    l_wait(0, 0)
    g_start(0, 0)
    l_wait(1, 1)
    g_start(1, 1)

    def tri(t, carry):
        for m in range(NBUF):
            j = 3 * t + m
            g_wait(j, m)

            @pl.when(j + 2 < CH)
            def _():
                l_wait(j + 2, (m + 2) % NBUF)
                g_start(j + 2, (m + 2) % NBUF)

            pltpu.sync_copy(bufs[m], acc_sh.at[idx_r.at[m, 1]], add=True)

            @pl.when(j + 3 < CH)
            def _():
                l_start(j + 3, m)

        return carry

    lax.fori_loop(0, CH // 3, tri, 0)
    plsc.subcore_barrier()

    @pl.when(c == 0)
    def _():
        pltpu.sync_copy(acc_sh.at[rows], p0_ref.at[rows])

    @pl.when(c == 1)
    def _():
        pltpu.sync_copy(acc_sh.at[rows], p1_ref.at[rows])


_mp_kernel = pl.kernel(
    _mp_body,
    out_type=(
        jax.ShapeDtypeStruct((NP, D), jnp.float32),
        jax.ShapeDtypeStruct((NP, D), jnp.float32),
    ),
    mesh=_MESH,
    scratch_types=[
        pltpu.VMEM((NBUF, 2, CB), jnp.int32),
        pltpu.VMEM((CB, D), jnp.float32),
        pltpu.VMEM((CB, D), jnp.float32),
        pltpu.VMEM((CB, D), jnp.float32),
        pltpu.VMEM_SHARED((NP, D), jnp.float32),
        pltpu.SemaphoreType.DMA,
        pltpu.SemaphoreType.DMA,
    ],
)


def _fin_body(p0_ref, p1_ref, nrm_ref, b_ref, out_ref):
    nrm = nrm_ref[:, 0:1]
    b_row = b_ref[...].reshape(1, D)
    out_ref[...] = (p0_ref[...] + p1_ref[...]) * nrm + b_row


_fin_kernel = pl.pallas_call(
    _fin_body,
    out_shape=jax.ShapeDtypeStruct((NP, D), jnp.float32),
)


@jax.jit
def kernel(x, edge_index, W, b):
    ei = edge_index.astype(jnp.int32)
    pad_idx = N + jnp.arange(EP - E, dtype=jnp.int32) % (NP - N)
    pad_blk = jnp.broadcast_to(pad_idx, (2, EP - E))
    ei = jnp.concatenate([ei, pad_blk], axis=1)
    ei = ei.reshape(2, NW, CH, CB)
    eint = jnp.transpose(ei, (1, 2, 0, 3))
    x_pad = jnp.pad(x, ((0, NP - N), (0, 0)))
    zeros = jnp.zeros((NP, D), jnp.float32)

    ei_k1 = ei.reshape(2, NS, 2 * CH, CB)
    hs, hd = _hist_kernel(ei_k1)
    y, nrm8 = _mm_kernel(x_pad, W, hs, hd)
    p0, p1 = _mp_kernel(eint, y, zeros)
    out_pad = _fin_kernel(p0, p1, nrm8, b)
    return out_pad[:N]


# speedup vs baseline: 1.0400x; 1.0400x over previous
import jax
import jax.numpy as jnp
from jax import lax
from jax.experimental import pallas as pl
from jax.experimental.pallas import tpu as pltpu
from jax.experimental.pallas import tpu_sc as plsc

N = 10000
E = 320000
D = 128
NC, NS = 2, 16
NW = NC * NS
CB = 128
CH = 80
HCH = 40
EP = NW * CH * CB
RPT = 640
NP = NS * RPT

_MESH = plsc.VectorSubcoreMesh(
    core_axis_name="c", subcore_axis_name="s", num_cores=NC, num_subcores=NS
)


def _hist_body(ei_ref, hs_ref, hd_ref, idx_v, hist_v):
    c = lax.axis_index("c")
    s = lax.axis_index("s")

    def zero(i, carry):
        hist_v[pl.ds(i * 16, 16)] = jnp.zeros((16,), jnp.float32)
        return carry

    lax.fori_loop(0, NP // 16, zero, 0)
    pltpu.sync_copy(ei_ref.at[c, s], idx_v)
    ones = jnp.ones((16,), jnp.float32)

    def chunk(j, carry):
        for k in range(CB // 16):
            idx16 = idx_v[j, pl.ds(k * 16, 16)]
            plsc.addupdate_scatter(hist_v, [idx16], ones)
        return carry

    lax.fori_loop(0, 2 * CH, chunk, 0)

    @pl.when(c == 0)
    def _():
        pltpu.sync_copy(hist_v, hs_ref.at[s])

    @pl.when(c == 1)
    def _():
        pltpu.sync_copy(hist_v, hd_ref.at[s])


_hist_kernel = pl.kernel(
    _hist_body,
    out_type=(
        jax.ShapeDtypeStruct((NS, NP), jnp.float32),
        jax.ShapeDtypeStruct((NS, NP), jnp.float32),
    ),
    mesh=_MESH,
    scratch_types=[
        pltpu.VMEM((2 * CH, CB), jnp.int32),
        pltpu.VMEM((NP,), jnp.float32),
    ],
    compiler_params=pltpu.CompilerParams(needs_layout_passes=False),
)


def _mm_body(x_ref, w_ref, hs_ref, hd_ref, y_ref, nrm_ref):
    ones_col = jnp.ones((NS, 1), jnp.float32)
    dn = (((0,), (0,)), ((), ()))
    outdeg = lax.dot_general(hs_ref[...], ones_col, dn,
                             preferred_element_type=jnp.float32)
    nsrc = lax.rsqrt(jnp.maximum(outdeg, 1.0))
    h = x_ref[...] * nsrc
    y_ref[...] = jnp.dot(h, w_ref[...], preferred_element_type=jnp.float32)
    indeg = lax.dot_general(hd_ref[...], ones_col, dn,
                            preferred_element_type=jnp.float32)
    ndst = lax.rsqrt(jnp.maximum(indeg, 1.0))
    nrm_ref[...] = jnp.broadcast_to(ndst, (NP, 8))


_mm_kernel = pl.pallas_call(
    _mm_body,
    out_shape=(
        jax.ShapeDtypeStruct((NP, D), jnp.float32),
        jax.ShapeDtypeStruct((NP, 8), jnp.float32),
    ),
)


def _mp_body(ei_ref, y_ref, zeros_ref, p0_ref, p1_ref,
             sidx, didx, rows_a, rows_b, acc_sh, sem):
    c = lax.axis_index("c")
    s = lax.axis_index("s")
    q = c * NS + s
    rows = pl.ds(s * RPT, RPT)
    pltpu.sync_copy(zeros_ref.at[rows], acc_sh.at[rows])
    plsc.subcore_barrier()

    bufs = (rows_a, rows_b)
    for h in range(CH // HCH):
        pltpu.sync_copy(ei_ref.at[0, q, pl.ds(h * HCH, HCH)], sidx)
        pltpu.sync_copy(ei_ref.at[1, q, pl.ds(h * HCH, HCH)], didx)
        pltpu.async_copy(y_ref.at[sidx.at[0]], rows_a, sem)

        def pair(g, carry):
            for bsel in range(2):
                j = 2 * g + bsel
                buf = bufs[bsel]

                @pl.when(j + 1 < HCH)
                def _():
                    pltpu.async_copy(
                        y_ref.at[sidx.at[j + 1]], bufs[1 - bsel], sem)

                pltpu.make_async_copy(y_ref.at[sidx.at[j]], buf, sem).wait()
                pltpu.sync_copy(buf, acc_sh.at[didx.at[j]], add=True)
            return carry

        lax.fori_loop(0, HCH // 2, pair, 0)
    plsc.subcore_barrier()

    @pl.when(c == 0)
    def _():
        pltpu.sync_copy(acc_sh.at[rows], p0_ref.at[rows])

    @pl.when(c == 1)
    def _():
        pltpu.sync_copy(acc_sh.at[rows], p1_ref.at[rows])


_mp_kernel = pl.kernel(
    _mp_body,
    out_type=(
        jax.ShapeDtypeStruct((NP, D), jnp.float32),
        jax.ShapeDtypeStruct((NP, D), jnp.float32),
    ),
    mesh=_MESH,
    scratch_types=[
        pltpu.VMEM((HCH, CB), jnp.int32),
        pltpu.VMEM((HCH, CB), jnp.int32),
        pltpu.VMEM((CB, D), jnp.float32),
        pltpu.VMEM((CB, D), jnp.float32),
        pltpu.VMEM_SHARED((NP, D), jnp.float32),
        pltpu.SemaphoreType.DMA,
    ],
)


def _fin_body(p0_ref, p1_ref, nrm_ref, b_ref, out_ref):
    nrm = nrm_ref[:, 0:1]
    b_row = b_ref[...].reshape(1, D)
    out_ref[...] = (p0_ref[...] + p1_ref[...]) * nrm + b_row


_fin_kernel = pl.pallas_call(
    _fin_body,
    out_shape=jax.ShapeDtypeStruct((NP, D), jnp.float32),
)


@jax.jit
def kernel(x, edge_index, W, b):
    ei = edge_index.astype(jnp.int32)
    pad_idx = N + jnp.arange(EP - E, dtype=jnp.int32) % (NP - N)
    pad_blk = jnp.broadcast_to(pad_idx, (2, EP - E))
    ei = jnp.concatenate([ei, pad_blk], axis=1)
    ei = ei.reshape(2, NW, CH, CB)
    x_pad = jnp.pad(x, ((0, NP - N), (0, 0)))
    zeros = jnp.zeros((NP, D), jnp.float32)

    ei_k1 = ei.reshape(2, NS, 2 * CH, CB)
    hs, hd = _hist_kernel(ei_k1)
    y, nrm8 = _mm_kernel(x_pad, W, hs, hd)
    p0, p1 = _mp_kernel(ei, y, zeros)
    out_pad = _fin_kernel(p0, p1, nrm8, b)
    return out_pad[:N]
